# baseline (speedup 1.0000x reference)
import functools

import jax
import jax.numpy as jnp
from jax import lax
from jax.experimental import pallas as pl
from jax.experimental.pallas import tpu as pltpu
from jax.experimental.pallas import tpu_sc as plsc

_EPS = 0.1
_V = 32000
_N = 2048
_IGNORE = 0
_CONF = 1.0 - _EPS
_EPS_U = _EPS / (_V - 1)

_BR = 256
_BV = 16000
_NR = _N // _BR
_NV = _V // _BV

_NC = 2
_NS = 16
_NW = _NC * _NS
_BPW = _N // _NW
_L = 16


def _sc_gather_kernel(x_ref, tgt_ref, out_ref, tgt_v, buf, g_v, sem):
    wid = lax.axis_index("s") * _NC + lax.axis_index("c")
    base = wid * _BPW
    pltpu.sync_copy(tgt_ref.at[pl.ds(base, _BPW)], tgt_v)

    lane_ids = lax.iota(jnp.int32, _L)
    for q in range(_BPW // _L):
        t16 = tgt_v[pl.ds(q * _L, _L)]
        for l in range(_L):
            k = q * _L + l
            t = jnp.sum(jnp.where(lane_ids == l, t16, 0))
            c0 = pl.multiple_of(
                lax.shift_left(lax.shift_right_logical(t, 7), 7), 128)
            r0 = pl.multiple_of(base + (k // 8) * 8, 8)
            pltpu.async_copy(x_ref.at[pl.ds(r0, 8), pl.ds(c0, 128)],
                             buf.at[k], sem)

    def _drain(k, carry):
        pltpu.make_async_copy(x_ref.at[pl.ds(0, 8), pl.ds(0, 128)],
                              buf.at[0], sem).wait()
        return carry

    lax.fori_loop(0, _BPW, _drain, 0)

    for q in range(_BPW // _L):
        t16 = tgt_v[pl.ds(q * _L, _L)]
        lanes = jnp.bitwise_and(t16, 127)
        k16 = q * _L + lax.iota(jnp.int32, _L)
        sub = jnp.bitwise_and(k16, 7)
        g_v[pl.ds(q * _L, _L)] = plsc.load_gather(buf, [k16, sub, lanes])

    pltpu.sync_copy(g_v, out_ref.at[pl.ds(base, _BPW)])


_sc_gather = functools.partial(
    pl.kernel,
    out_type=jax.ShapeDtypeStruct((_N,), jnp.float32),
    mesh=plsc.VectorSubcoreMesh(core_axis_name="c", subcore_axis_name="s"),
    scratch_types=[
        pltpu.VMEM((_BPW,), jnp.int32),
        pltpu.VMEM((_BPW, 8, 128), jnp.float32),
        pltpu.VMEM((_BPW,), jnp.float32),
        pltpu.SemaphoreType.DMA,
    ],
    compiler_params=pltpu.CompilerParams(use_tc_tiling_on_sc=True,
                                         needs_layout_passes=False),
)(_sc_gather_kernel)


def _loss_kernel(x_ref, tgt_ref, g_ref, out_ref, m_ref, s_ref, t_ref):
    i = pl.program_id(0)
    j = pl.program_id(1)

    x = x_ref[...]

    @pl.when(j == 0)
    def _init():
        m_ref[...] = jnp.full((_BR, 1), -jnp.inf, jnp.float32)
        s_ref[...] = jnp.zeros((_BR, 1), jnp.float32)
        t_ref[...] = jnp.zeros((_BR, 1), jnp.float32)

    m_old = m_ref[...]
    blk_max = jnp.max(x, axis=1, keepdims=True)
    m_new = jnp.maximum(m_old, blk_max)
    ones = jnp.ones((_BV, 1), jnp.float32)
    e = jnp.exp(x - m_new)
    s_ref[...] = (s_ref[...] * jnp.exp(m_old - m_new)
                  + jax.lax.dot_general(
                      e, ones, (((1,), (0,)), ((), ())),
                      preferred_element_type=jnp.float32))
    m_ref[...] = m_new
    t_ref[...] = t_ref[...] + jax.lax.dot_general(
        x, ones, (((1,), (0,)), ((), ())),
        preferred_element_type=jnp.float32)

    @pl.when(j == _NV - 1)
    def _finish():
        tgt = tgt_ref[i]
        g = g_ref[i]
        lse = m_ref[...] + jnp.log(s_ref[...])
        gp = g - lse
        loss_rows = (_EPS_U * (_V * lse - t_ref[...])
                     - (_CONF - _EPS_U) * gp)
        loss_rows = jnp.where(tgt == _IGNORE, 0.0, loss_rows)
        part = jnp.sum(loss_rows) * (1.0 / _N)

        @pl.when(i == 0)
        def _first():
            out_ref[0, 0] = part

        @pl.when(i > 0)
        def _rest():
            out_ref[0, 0] = out_ref[0, 0] + part


_tc_loss = pl.pallas_call(
    _loss_kernel,
    grid=(_NR, _NV),
    in_specs=[
        pl.BlockSpec((_BR, _BV), lambda i, j: (i, j)),
        pl.BlockSpec((_NR, _BR, 1), lambda i, j: (0, 0, 0)),
        pl.BlockSpec((_NR, _BR, 1), lambda i, j: (0, 0, 0)),
    ],
    out_specs=pl.BlockSpec((1, 1), lambda i, j: (0, 0),
                           memory_space=pltpu.SMEM),
    out_shape=jax.ShapeDtypeStruct((1, 1), jnp.float32),
    scratch_shapes=[
        pltpu.VMEM((_BR, 1), jnp.float32),
        pltpu.VMEM((_BR, 1), jnp.float32),
        pltpu.VMEM((_BR, 1), jnp.float32),
    ],
)


@jax.jit
def kernel(output, target):
    g = _sc_gather(output, target)
    out = _tc_loss(output,
                   target.reshape(_NR, _BR, 1),
                   g.reshape(_NR, _BR, 1))
    return out[0, 0]


# speedup vs baseline: 1.0588x; 1.0588x over previous
import functools

import jax
import jax.numpy as jnp
from jax import lax
from jax.experimental import pallas as pl
from jax.experimental.pallas import tpu as pltpu
from jax.experimental.pallas import tpu_sc as plsc

_EPS = 0.1
_V = 32000
_N = 2048
_IGNORE = 0
_CONF = 1.0 - _EPS
_EPS_U = _EPS / (_V - 1)

_BR = 256
_BV = 16000
_NR = _N // _BR
_NV = _V // _BV

_NC = 2
_NS = 16
_NW = _NC * _NS
_BPW = _N // _NW
_L = 16


def _sc_gather_kernel(x_ref, tgt_ref, out_ref, tgt_v, buf, g_v, sem):
    wid = lax.axis_index("s") * _NC + lax.axis_index("c")
    base = wid * _BPW
    pltpu.sync_copy(tgt_ref.at[pl.ds(base, _BPW)], tgt_v)

    lane_ids = lax.iota(jnp.int32, _L)
    for q in range(_BPW // _L):
        t16 = tgt_v[pl.ds(q * _L, _L)]
        for l in range(_L):
            k = q * _L + l
            t = jnp.sum(jnp.where(lane_ids == l, t16, 0))
            c0 = pl.multiple_of(
                lax.shift_left(lax.shift_right_logical(t, 7), 7), 128)
            r0 = pl.multiple_of(base + (k // 8) * 8, 8)
            pltpu.async_copy(x_ref.at[pl.ds(r0, 8), pl.ds(c0, 128)],
                             buf.at[k], sem)

    def _drain(k, carry):
        pltpu.make_async_copy(x_ref.at[pl.ds(0, 8), pl.ds(0, 128)],
                              buf.at[0], sem).wait()
        return carry

    lax.fori_loop(0, _BPW, _drain, 0)

    for q in range(_BPW // _L):
        t16 = tgt_v[pl.ds(q * _L, _L)]
        lanes = jnp.bitwise_and(t16, 127)
        k16 = q * _L + lax.iota(jnp.int32, _L)
        sub = jnp.bitwise_and(k16, 7)
        g_v[pl.ds(q * _L, _L)] = plsc.load_gather(buf, [k16, sub, lanes])

    pltpu.sync_copy(g_v, out_ref.at[pl.ds(base, _BPW)])


_sc_gather = functools.partial(
    pl.kernel,
    out_type=jax.ShapeDtypeStruct((_N,), jnp.float32),
    mesh=plsc.VectorSubcoreMesh(core_axis_name="c", subcore_axis_name="s"),
    scratch_types=[
        pltpu.VMEM((_BPW,), jnp.int32),
        pltpu.VMEM((_BPW, 8, 128), jnp.float32),
        pltpu.VMEM((_BPW,), jnp.float32),
        pltpu.SemaphoreType.DMA,
    ],
    compiler_params=pltpu.CompilerParams(use_tc_tiling_on_sc=True,
                                         needs_layout_passes=False),
)(_sc_gather_kernel)


def _loss_kernel(x_ref, tgt_ref, g_ref, out_ref, m_ref, s_ref, t_ref):
    i = pl.program_id(0)
    j = pl.program_id(1)

    x = x_ref[...]

    @pl.when(j == 0)
    def _init():
        m_ref[...] = jnp.full((_BR, 1), -jnp.inf, jnp.float32)
        s_ref[...] = jnp.zeros((_BR, 1), jnp.float32)
        t_ref[...] = jnp.zeros((_BR, 1), jnp.float32)

    m_old = m_ref[...]
    blk_max = jnp.max(x, axis=1, keepdims=True)
    m_new = jnp.maximum(m_old, blk_max)
    ones = jnp.ones((_BV, 1), jnp.float32)
    s_ref[...] = (s_ref[...] * jnp.exp(m_old - m_new)
                  + jnp.sum(jnp.exp(x - m_new), axis=1, keepdims=True))
    m_ref[...] = m_new
    t_ref[...] = t_ref[...] + jax.lax.dot_general(
        x, ones, (((1,), (0,)), ((), ())),
        preferred_element_type=jnp.float32)

    @pl.when(j == _NV - 1)
    def _finish():
        tgt = tgt_ref[i]
        g = g_ref[i]
        lse = m_ref[...] + jnp.log(s_ref[...])
        gp = g - lse
        loss_rows = (_EPS_U * (_V * lse - t_ref[...])
                     - (_CONF - _EPS_U) * gp)
        loss_rows = jnp.where(tgt == _IGNORE, 0.0, loss_rows)
        part = jnp.sum(loss_rows) * (1.0 / _N)

        @pl.when(i == 0)
        def _first():
            out_ref[0, 0] = part

        @pl.when(i > 0)
        def _rest():
            out_ref[0, 0] = out_ref[0, 0] + part


_tc_loss = pl.pallas_call(
    _loss_kernel,
    grid=(_NR, _NV),
    in_specs=[
        pl.BlockSpec((_BR, _BV), lambda i, j: (i, j)),
        pl.BlockSpec((_NR, _BR, 1), lambda i, j: (0, 0, 0)),
        pl.BlockSpec((_NR, _BR, 1), lambda i, j: (0, 0, 0)),
    ],
    out_specs=pl.BlockSpec((1, 1), lambda i, j: (0, 0),
                           memory_space=pltpu.SMEM),
    out_shape=jax.ShapeDtypeStruct((1, 1), jnp.float32),
    scratch_shapes=[
        pltpu.VMEM((_BR, 1), jnp.float32),
        pltpu.VMEM((_BR, 1), jnp.float32),
        pltpu.VMEM((_BR, 1), jnp.float32),
    ],
)


@jax.jit
def kernel(output, target):
    g = _sc_gather(output, target)
    out = _tc_loss(output,
                   target.reshape(_NR, _BR, 1),
                   g.reshape(_NR, _BR, 1))
    return out[0, 0]


# speedup vs baseline: 1.2250x; 1.1569x over previous
import functools

import jax
import jax.numpy as jnp
from jax import lax
from jax.experimental import pallas as pl
from jax.experimental.pallas import tpu as pltpu
from jax.experimental.pallas import tpu_sc as plsc

_EPS = 0.1
_V = 32000
_N = 2048
_IGNORE = 0
_CONF = 1.0 - _EPS
_EPS_U = _EPS / (_V - 1)

_BR = 256
_BV = 16000
_NR = _N // _BR
_NV = _V // _BV

_NC = 2
_NS = 16
_NW = _NC * _NS
_BPW = _N // _NW
_L = 16


def _sc_gather_kernel(x_ref, tgt_ref, out_ref, tgt_v, buf, g_v, sem):
    wid = lax.axis_index("s") * _NC + lax.axis_index("c")
    base = wid * _BPW
    pltpu.sync_copy(tgt_ref.at[pl.ds(base, _BPW)], tgt_v)

    lane_ids = lax.iota(jnp.int32, _L)
    for q in range(_BPW // _L):
        t16 = tgt_v[pl.ds(q * _L, _L)]
        for l in range(_L):
            k = q * _L + l
            t = jnp.sum(jnp.where(lane_ids == l, t16, 0))
            c0 = pl.multiple_of(
                lax.shift_left(lax.shift_right_logical(t, 7), 7), 128)
            r0 = pl.multiple_of(base + (k // 8) * 8, 8)
            pltpu.async_copy(x_ref.at[pl.ds(r0, 8), pl.ds(c0, 128)],
                             buf.at[k], sem)

    def _drain(k, carry):
        pltpu.make_async_copy(x_ref.at[pl.ds(0, 8), pl.ds(0, 128)],
                              buf.at[0], sem).wait()
        return carry

    lax.fori_loop(0, _BPW, _drain, 0)

    for q in range(_BPW // _L):
        t16 = tgt_v[pl.ds(q * _L, _L)]
        lanes = jnp.bitwise_and(t16, 127)
        k16 = q * _L + lax.iota(jnp.int32, _L)
        sub = jnp.bitwise_and(k16, 7)
        g_v[pl.ds(q * _L, _L)] = plsc.load_gather(buf, [k16, sub, lanes])

    pltpu.sync_copy(g_v, out_ref.at[pl.ds(base, _BPW)])


_sc_gather = functools.partial(
    pl.kernel,
    out_type=jax.ShapeDtypeStruct((_N,), jnp.float32),
    mesh=plsc.VectorSubcoreMesh(core_axis_name="c", subcore_axis_name="s"),
    scratch_types=[
        pltpu.VMEM((_BPW,), jnp.int32),
        pltpu.VMEM((_BPW, 8, 128), jnp.float32),
        pltpu.VMEM((_BPW,), jnp.float32),
        pltpu.SemaphoreType.DMA,
    ],
    compiler_params=pltpu.CompilerParams(use_tc_tiling_on_sc=True,
                                         needs_layout_passes=False),
)(_sc_gather_kernel)


def _loss_kernel(x_ref, tgt_ref, g_ref, out_ref, m_ref, s_ref, t_ref):
    i = pl.program_id(0)
    j = pl.program_id(1)

    @pl.when(j == 0)
    def _init():
        m_ref[...] = jnp.full((_BR, 1), -jnp.inf, jnp.float32)
        s_ref[...] = jnp.zeros((_BR, 1), jnp.float32)
        t_ref[...] = jnp.zeros((_BR, 1), jnp.float32)

    m_old = m_ref[...]
    blk_max = jnp.max(x_ref[...], axis=1, keepdims=True)
    m_new = jnp.maximum(m_old, blk_max)
    s_ref[...] = (s_ref[...] * jnp.exp(m_old - m_new)
                  + jnp.sum(jnp.exp(x_ref[...] - m_new), axis=1,
                            keepdims=True))
    m_ref[...] = m_new
    t_ref[...] = t_ref[...] + jnp.sum(x_ref[...], axis=1, keepdims=True)

    @pl.when(j == _NV - 1)
    def _finish():
        tgt = tgt_ref[i]
        g = g_ref[i]
        lse = m_ref[...] + jnp.log(s_ref[...])
        gp = g - lse
        loss_rows = (_EPS_U * (_V * lse - t_ref[...])
                     - (_CONF - _EPS_U) * gp)
        loss_rows = jnp.where(tgt == _IGNORE, 0.0, loss_rows)
        part = jnp.sum(loss_rows) * (1.0 / _N)

        @pl.when(i == 0)
        def _first():
            out_ref[0, 0] = part

        @pl.when(i > 0)
        def _rest():
            out_ref[0, 0] = out_ref[0, 0] + part


_tc_loss = pl.pallas_call(
    _loss_kernel,
    grid=(_NR, _NV),
    in_specs=[
        pl.BlockSpec((_BR, _BV), lambda i, j: (i, j)),
        pl.BlockSpec((_NR, _BR, 1), lambda i, j: (0, 0, 0)),
        pl.BlockSpec((_NR, _BR, 1), lambda i, j: (0, 0, 0)),
    ],
    out_specs=pl.BlockSpec((1, 1), lambda i, j: (0, 0),
                           memory_space=pltpu.SMEM),
    out_shape=jax.ShapeDtypeStruct((1, 1), jnp.float32),
    scratch_shapes=[
        pltpu.VMEM((_BR, 1), jnp.float32),
        pltpu.VMEM((_BR, 1), jnp.float32),
        pltpu.VMEM((_BR, 1), jnp.float32),
    ],
)


@jax.jit
def kernel(output, target):
    g = _sc_gather(output, target)
    out = _tc_loss(output,
                   target.reshape(_NR, _BR, 1),
                   g.reshape(_NR, _BR, 1))
    return out[0, 0]
